# baseline (speedup 1.0000x reference)
import functools

import jax
import jax.numpy as jnp
from jax import lax
from jax.experimental import pallas as pl
from jax.experimental.pallas import tpu as pltpu
from jax.experimental.pallas import tpu_sc as plsc

DIM = 1024
MAX_T = 1000
BATCH = 16384

NC = 2
NS = 16
NW = NC * NS
BPW = BATCH // NW
CHUNK_SIZES = [64, 56, 64, 56, 64, 56, 64, 56, 32]
BUF_SIZES = [64, 56]
NCHUNK = len(CHUNK_SIZES)
CHUNK_OFFS = [sum(CHUNK_SIZES[:i]) for i in range(NCHUNK)]
NBUF = 2


def _gather_body(table_hbm, idx_hbm, out_hbm, idx_v, *bufs_and_sems):
    bufs = bufs_and_sems[:NBUF]
    gsems = bufs_and_sems[NBUF:2 * NBUF]
    osems = bufs_and_sems[2 * NBUF:3 * NBUF]

    cid = lax.axis_index("c")
    sid = lax.axis_index("s")
    wid = sid * NC + cid
    base = wid * BPW

    pltpu.sync_copy(idx_hbm.at[pl.ds(base, BPW)], idx_v)

    g_handles = [None] * NCHUNK
    o_handles = [None] * NBUF

    for j in range(NCHUNK + 1):
        b = j % NBUF
        if j < NCHUNK:
            if j >= NBUF:
                o_handles[b].wait()
            n = CHUNK_SIZES[j]
            dst = bufs[b] if n == BUF_SIZES[b] else bufs[b].at[pl.ds(0, n)]
            g_handles[j] = pltpu.async_copy(
                table_hbm.at[idx_v.at[pl.ds(CHUNK_OFFS[j], n)]],
                dst, gsems[b])
        if j >= 1:
            pb = (j - 1) % NBUF
            g_handles[j - 1].wait()
            pn = CHUNK_SIZES[j - 1]
            src = bufs[pb] if pn == BUF_SIZES[pb] else bufs[pb].at[pl.ds(0, pn)]
            o_handles[pb] = pltpu.async_copy(
                src, out_hbm.at[pl.ds(base + CHUNK_OFFS[j - 1], pn)],
                osems[pb])
    for h in o_handles:
        h.wait()


@jax.jit
def _pe_lookup(table, idx3):
    mesh = plsc.VectorSubcoreMesh(core_axis_name="c", subcore_axis_name="s")
    k = functools.partial(
        pl.kernel,
        mesh=mesh,
        out_type=jax.ShapeDtypeStruct((BATCH, 1, DIM), jnp.float32),
        scratch_types=(
            [pltpu.VMEM((BPW,), jnp.int32)]
            + [pltpu.VMEM((n, 1, DIM), jnp.float32) for n in BUF_SIZES]
            + [pltpu.SemaphoreType.DMA for _ in range(2 * NBUF)]
        ),
    )(_gather_body)
    return k(table, idx3)


def kernel(timestep, pe_matrix):
    return _pe_lookup(pe_matrix, timestep.astype(jnp.int32))


# speedup vs baseline: 1.0074x; 1.0074x over previous
import functools

import jax
import jax.numpy as jnp
from jax import lax
from jax.experimental import pallas as pl
from jax.experimental.pallas import tpu as pltpu
from jax.experimental.pallas import tpu_sc as plsc

DIM = 1024
MAX_T = 1000
BATCH = 16384

NC = 2
NS = 16
NW = NC * NS
BPW = BATCH // NW
CHUNK_SIZES = [48] * 10 + [32]
BUF_SIZES = [48, 48]
NCHUNK = len(CHUNK_SIZES)
CHUNK_OFFS = [sum(CHUNK_SIZES[:i]) for i in range(NCHUNK)]
NBUF = 2


def _gather_body(table_hbm, idx_hbm, out_hbm, idx_v, *bufs_and_sems):
    bufs = bufs_and_sems[:NBUF]
    gsems = bufs_and_sems[NBUF:2 * NBUF]
    osems = bufs_and_sems[2 * NBUF:3 * NBUF]

    cid = lax.axis_index("c")
    sid = lax.axis_index("s")
    wid = sid * NC + cid
    base = wid * BPW

    pltpu.sync_copy(idx_hbm.at[pl.ds(base, BPW)], idx_v)

    g_handles = [None] * NCHUNK
    o_handles = [None] * NBUF

    for j in range(NCHUNK + 1):
        b = j % NBUF
        if j < NCHUNK:
            if j >= NBUF:
                o_handles[b].wait()
            n = CHUNK_SIZES[j]
            dst = bufs[b] if n == BUF_SIZES[b] else bufs[b].at[pl.ds(0, n)]
            g_handles[j] = pltpu.async_copy(
                table_hbm.at[idx_v.at[pl.ds(CHUNK_OFFS[j], n)]],
                dst, gsems[b])
        if j >= 1:
            pb = (j - 1) % NBUF
            g_handles[j - 1].wait()
            pn = CHUNK_SIZES[j - 1]
            src = bufs[pb] if pn == BUF_SIZES[pb] else bufs[pb].at[pl.ds(0, pn)]
            o_handles[pb] = pltpu.async_copy(
                src, out_hbm.at[pl.ds(base + CHUNK_OFFS[j - 1], pn)],
                osems[pb])
    for h in o_handles:
        h.wait()


@jax.jit
def _pe_lookup(table, idx3):
    mesh = plsc.VectorSubcoreMesh(core_axis_name="c", subcore_axis_name="s")
    k = functools.partial(
        pl.kernel,
        mesh=mesh,
        out_type=jax.ShapeDtypeStruct((BATCH, 1, DIM), jnp.float32),
        scratch_types=(
            [pltpu.VMEM((BPW,), jnp.int32)]
            + [pltpu.VMEM((n, 1, DIM), jnp.float32) for n in BUF_SIZES]
            + [pltpu.SemaphoreType.DMA for _ in range(2 * NBUF)]
        ),
    )(_gather_body)
    return k(table, idx3)


def kernel(timestep, pe_matrix):
    return _pe_lookup(pe_matrix, timestep.astype(jnp.int32))
